# baseline (speedup 1.0000x reference)
import functools

import jax
import jax.numpy as jnp
from jax import lax
from jax.experimental import pallas as pl
from jax.experimental.pallas import tpu as pltpu
from jax.experimental.pallas import tpu_sc as plsc

B, D = 16384, 64
NC, NS, L = 2, 16, 16
NW = NC * NS
RPW = B // NW
CH = 128
NCH = RPW // CH
UNROLL = 4


def _sc_center_loss(feats, label2d, centers):
    mesh = plsc.VectorSubcoreMesh(core_axis_name="c", subcore_axis_name="s")

    @functools.partial(
        pl.kernel,
        mesh=mesh,
        compiler_params=pltpu.CompilerParams(use_tc_tiling_on_sc=False),
        out_type=jax.ShapeDtypeStruct((NW, L), jnp.float32),
        scratch_types=[
            pltpu.VMEM((NCH, CH), jnp.int32),
            pltpu.VMEM((RPW, D), jnp.float32),
            pltpu.VMEM((RPW, D), jnp.float32),
            pltpu.VMEM((L,), jnp.float32),
            pltpu.SemaphoreType.DMA,
        ],
    )
    def k(feats_hbm, label_hbm, centers_hbm, out_hbm, idx_v, f_v, c_v, acc_v, sem):
        wid = lax.axis_index("s") * NC + lax.axis_index("c")

        pltpu.sync_copy(label_hbm.at[pl.ds(wid * NCH, NCH)], idx_v)
        copies = [
            pltpu.async_copy(
                centers_hbm.at[idx_v.at[kk]],
                c_v.at[pl.ds(kk * CH, CH)],
                sem,
            )
            for kk in range(NCH)
        ]
        pltpu.sync_copy(feats_hbm.at[pl.ds(wid * RPW, RPW)], f_v)
        for c in copies:
            c.wait()

        lane = lax.iota(jnp.int32, 16)
        is_last = lane == 15
        zero = jnp.zeros((L,), jnp.float32)
        perms = [lane ^ sh for sh in (8, 4, 2, 1)]

        def body(i, acc):
            for u in range(UNROLL):
                r = i * UNROLL + u
                s = None
                for cc in range(D // L):
                    df = f_v[r, pl.ds(cc * L, L)] - c_v[r, pl.ds(cc * L, L)]
                    sq = df * df
                    s = sq if s is None else s + sq
                for p in perms:
                    s = s + s.at[p].get(mode="promise_in_bounds")
                w = jnp.clip(s * 0.5, 1e-12, 1e12)
                acc = acc + jnp.where(is_last, w, zero)
            return acc

        acc = lax.fori_loop(0, RPW // UNROLL, body, zero)
        acc_v[...] = acc
        pltpu.sync_copy(acc_v, out_hbm.at[wid])

    return k(feats, label2d, centers)


def kernel(feats, label, centers):
    label2d = label.reshape(B // CH, CH)
    partials = _sc_center_loss(feats, label2d, centers)
    return jnp.sum(partials) / 16384.0
